# baseline (speedup 1.0000x reference)
import functools

import jax
import jax.numpy as jnp
from jax import lax
from jax.experimental import pallas as pl
from jax.experimental.pallas import tpu as pltpu
from jax.experimental.pallas import tpu_sc as plsc

NC = 2
NS = 16
NT = NC * NS
B = 128
S = 512
RPO = 320
N_PAD = NT * RPO
HALF = N_PAD // NC


def _bucket_kernel(ept):
    mesh = plsc.VectorSubcoreMesh(core_axis_name="c", subcore_axis_name="s")

    @functools.partial(
        pl.kernel,
        out_type=(jax.ShapeDtypeStruct((NT, NT, S), jnp.int32),
                  jax.ShapeDtypeStruct((NT, NT, S), jnp.int32),
                  jax.ShapeDtypeStruct((NT, NT), jnp.int32)),
        mesh=mesh,
        compiler_params=pltpu.CompilerParams(use_tc_tiling_on_sc=False,
                                             needs_layout_passes=False),
        scratch_types=[
            pltpu.VMEM((ept,), jnp.int32),
            pltpu.VMEM((ept,), jnp.int32),
            pltpu.VMEM((ept,), jnp.int32),
            pltpu.VMEM((NT * S + 16,), jnp.int32),
            pltpu.VMEM((NT * S + 16,), jnp.int32),
            pltpu.VMEM((48,), jnp.int32),
        ],
    )
    def bucket(srcs, dsts, bsrc, bdst, counts, sv, dv, ov, ls, ld, cvm):
        g = lax.axis_index("c") * NS + lax.axis_index("s")
        pltpu.sync_copy(srcs.at[g], sv)
        pltpu.sync_copy(dsts.at[g], dv)

        def fill(i, carry):
            o = i // (S // 16)
            padv = jnp.where(o >= NS, NC * HALF, HALF)
            ls[pl.ds(i * 16, 16)] = jnp.zeros((16,), jnp.int32)
            ld[pl.ds(i * 16, 16)] = jnp.full((16,), padv, jnp.int32)
            return carry

        lax.fori_loop(0, NT * S // 16, fill, 0)

        def owners(v, carry):
            dvec = dv[pl.ds(v * 16, 16)]
            ov[pl.ds(v * 16, 16)] = (dvec * 6554) >> 21
            return carry

        lax.fori_loop(0, ept // 16, owners, 0)

        for o in range(NT):
            o_base = o * S

            def scan(v, pos, o=o, o_base=o_base):
                msk_i = jnp.where(ov[pl.ds(v * 16, 16)] == o, 1, 0)
                rank = plsc.cumsum(msk_i) - msk_i
                idx = jnp.minimum(o_base + pos + rank, o_base + S - 1)
                idx = jnp.where(msk_i == 1, idx, NT * S)
                plsc.store_scatter(ls, [idx], sv[pl.ds(v * 16, 16)])
                plsc.store_scatter(ld, [idx], dv[pl.ds(v * 16, 16)])
                return pos + jnp.sum(msk_i)

            pos = lax.fori_loop(0, ept // 16, scan, 0)
            plsc.store_scatter(cvm, [jnp.full((16,), o, jnp.int32)],
                               jnp.full((16,), pos, jnp.int32))

        pltpu.sync_copy(cvm.at[pl.ds(0, NT)], counts.at[g])
        for o in range(NT):
            pltpu.sync_copy(ls.at[pl.ds(o * S, S)], bsrc.at[o, g])
            pltpu.sync_copy(ld.at[pl.ds(o * S, S)], bdst.at[o, g])

    return bucket


def _make_segsum(w):
    mesh = plsc.VectorSubcoreMesh(core_axis_name="c", subcore_axis_name="s")

    @functools.partial(
        pl.kernel,
        out_type=jax.ShapeDtypeStruct((N_PAD, w), jnp.float32),
        mesh=mesh,
        compiler_params=pltpu.CompilerParams(use_tc_tiling_on_sc=False),
        scratch_types=[
            pltpu.VMEM((4, B), jnp.int32),
            pltpu.VMEM((4, B), jnp.int32),
            pltpu.VMEM((B, w), jnp.float32),
            pltpu.VMEM((NT * NT + 16,), jnp.int32),
            pltpu.VMEM_SHARED((HALF + 8, w), jnp.float32),
        ],
    )
    def seg(table, bsrc, bdst, counts, zeros, out, sbuf, dbuf, rows,
            cvm, acc):
        cid = lax.axis_index("c")
        sid = lax.axis_index("s")
        o = cid * NS + sid
        base = cid * HALF

        pltpu.sync_copy(counts, cvm.at[pl.ds(0, NT * NT)])

        pltpu.sync_copy(zeros.at[pl.ds(0, RPO)],
                        acc.at[pl.ds(sid * RPO, RPO)])
        pl.when(sid == 0)(
            lambda: pltpu.sync_copy(zeros.at[pl.ds(0, 8)],
                                    acc.at[pl.ds(HALF, 8)]))
        plsc.subcore_barrier()

        bvec = jnp.full((16,), base, jnp.int32)

        def chunk(c, t2):
            pltpu.sync_copy(bsrc.at[o, t2, pl.ds(c * B, B)], sbuf.at[0])
            pltpu.sync_copy(bdst.at[o, t2, pl.ds(c * B, B)], dbuf.at[0])
            for j in range(B // 16):
                dbuf[0, pl.ds(j * 16, 16)] = (
                    dbuf[0, pl.ds(j * 16, 16)] - bvec)
            pltpu.sync_copy(table.at[sbuf.at[0]], rows)
            pltpu.sync_copy(rows, acc.at[dbuf.at[0]], add=True)
            return t2

        def per_tile(t2, carry):
            cntv = cvm[pl.ds(t2 * NT + o, 16)][0]
            nch = (cntv + (B - 1)) >> 7
            lax.fori_loop(0, nch, chunk, t2)
            return carry

        lax.fori_loop(0, NT, per_tile, 0)
        plsc.subcore_barrier()

        pltpu.sync_copy(acc.at[pl.ds(sid * RPO, RPO)],
                        out.at[pl.ds(base + sid * RPO, RPO)])

    return seg


def _tc_stage(n, dout, nagg, relu):
    def body(*refs):
        aggs = refs[:nagg]
        h, wrelT, brel, wrootT, gamma, beta, hout = refs[nagg:]
        agg = jnp.concatenate([a[:n, :] for a in aggs], axis=1)
        o = (jnp.dot(agg, wrelT[...], preferred_element_type=jnp.float32)
             + brel[...]
             + jnp.dot(h[...], wrootT[...],
                       preferred_element_type=jnp.float32))
        m = jnp.mean(o, axis=0, keepdims=True)
        v = jnp.mean((o - m) ** 2, axis=0, keepdims=True)
        o = (o - m) * lax.rsqrt(v + 1e-5) * gamma[...] + beta[...]
        if relu:
            o = jnp.maximum(o, 0.0)
        hout[...] = o

    return pl.pallas_call(
        body, out_shape=jax.ShapeDtypeStruct((n, dout), jnp.float32))


def kernel(x, edge_index, params):
    n, d_in = x.shape
    e = edge_index.shape[1]
    src, dst = edge_index[0], edge_index[1]
    assert e % NT == 0 and n + 1 <= N_PAD

    ept_real = e // NT
    ept = -(-(ept_real + 1) // 8) * 8 + 232
    padw = ept - ept_real
    srcs = jnp.concatenate(
        [src.reshape(NT, ept_real),
         jnp.zeros((NT, padw), jnp.int32)], axis=1)
    dsts = jnp.concatenate(
        [dst.reshape(NT, ept_real),
         jnp.full((NT, padw), n, jnp.int32)], axis=1)

    bsrc, bdst, counts = _bucket_kernel(ept)(srcs, dsts)
    counts_flat = counts.reshape(NT * NT)

    dims = [(wr.shape[1], wr.shape[0]) for wr, _, _, _, _ in params]
    widths = sorted({min(din, 128) for din, _ in dims})
    zeros_by_w = {w: jnp.zeros((RPO + 8, w), jnp.float32) for w in widths}
    segs = {w: _make_segsum(w) for w in widths}

    h = x
    for i, (wrel, brel, wroot, gamma, beta) in enumerate(params):
        din, dout = dims[i]
        w = min(din, 128)
        tables = [h] if din <= 128 else [
            h[:, j * 128:(j + 1) * 128] for j in range(din // 128)]
        aggs = [segs[w](t, bsrc, bdst, counts_flat, zeros_by_w[w])
                for t in tables]
        stage = _tc_stage(n, dout, len(aggs), relu=(i < len(params) - 1))
        h = stage(*aggs, h, wrel.T, brel, wroot.T, gamma, beta)
    return h
